# baseline (speedup 1.0000x reference)
import functools

import numpy as np

import jax
import jax.numpy as jnp
from jax import lax
from jax.experimental import pallas as pl
from jax.experimental.pallas import tpu as pltpu
from jax.experimental.pallas import tpu_sc as plsc

C = 384
H = 512
W = 512
N = H * W
NW = 32
G = 16
R = 16
QS = N // G
QD = N // R

_CONSTS = None


def _threefry2x32(k0, k1, x0, x1):
    u32 = np.uint32
    ks = [u32(k0), u32(k1), u32(k0) ^ u32(k1) ^ u32(0x1BD11BDA)]
    x = [x0.astype(u32).copy(), x1.astype(u32).copy()]

    def rounds(x, rots):
        for r in rots:
            x[0] = x[0] + x[1]
            x[1] = (x[1] << u32(r)) | (x[1] >> u32(32 - r))
            x[1] = x[0] ^ x[1]
        return x

    rot0, rot1 = [13, 15, 26, 6], [17, 29, 16, 24]
    x[0] = x[0] + ks[0]
    x[1] = x[1] + ks[1]
    x = rounds(x, rot0)
    x[0] = x[0] + ks[1]
    x[1] = x[1] + ks[2] + u32(1)
    x = rounds(x, rot1)
    x[0] = x[0] + ks[2]
    x[1] = x[1] + ks[0] + u32(2)
    x = rounds(x, rot0)
    x[0] = x[0] + ks[0]
    x[1] = x[1] + ks[1] + u32(3)
    x = rounds(x, rot1)
    x[0] = x[0] + ks[1]
    x[1] = x[1] + ks[2] + u32(4)
    x = rounds(x, rot0)
    x[0] = x[0] + ks[2]
    x[1] = x[1] + ks[0] + u32(5)
    return x[0], x[1]


def _np_permutation(seed, n):
    u32 = np.uint32
    key = (u32(np.int64(seed) >> 32), u32(np.int64(seed) & 0xFFFFFFFF))
    x = np.arange(n, dtype=np.int64)
    num_rounds = int(np.ceil(3 * np.log(max(1, n)) / np.log(2**32 - 1)))
    for _ in range(num_rounds):
        b1, b2 = _threefry2x32(
            key[0], key[1], np.zeros(2, u32), np.arange(2, dtype=u32)
        )
        key, subkey = (b1[0], b2[0]), (b1[1], b2[1])
        c1, c2 = np.zeros(n, u32), np.arange(n, dtype=u32)
        s1, s2 = _threefry2x32(subkey[0], subkey[1], c1, c2)
        x = x[np.argsort(s1 ^ s2, kind="stable")]
    return x


def _pack_pairs(idx):
    m, q = idx.shape
    pairs = idx.reshape(m, q // 32, 2, 16)
    return (pairs[:, :, 0, :] | (pairs[:, :, 1, :] << 16)).astype(
        np.int32
    ).reshape(m, q // 2)


def _consts():
    global _CONSTS
    if _CONSTS is None:
        perm = _np_permutation(0, N)
        inv = np.empty(N, np.int64)
        inv[perm] = np.arange(N)
        g_of = np.arange(N) // QS
        r_of = inv // QD
        key = g_of * R + r_of
        counts = np.bincount(key, minlength=G * R)
        starts = np.concatenate(([0], np.cumsum(counts)[:-1]))
        order = np.argsort(key, kind="stable")
        ranks = np.empty(N, np.int64)
        ranks[order] = np.arange(N) - np.repeat(starts, counts)
        pad = int(-(-counts.max() // 32) * 32)
        assert R * pad < 65536 and G * pad < 65536
        rho = np.zeros((G, R * pad), np.int64)
        rho[g_of, r_of * pad + ranks] = np.arange(N) % QS
        sigma = (g_of[perm] * pad + ranks[perm]).reshape(R, QD)
        _CONSTS = (pad, _pack_pairs(rho), _pack_pairs(sigma))
    return _CONSTS


_CPARAMS = dict(
    compiler_params=pltpu.CompilerParams(needs_layout_passes=False),
)


def _make_pass1(pad):
    mesh = plsc.VectorSubcoreMesh(core_axis_name="c", subcore_axis_name="s")

    @functools.partial(
        pl.kernel,
        mesh=mesh,
        out_type=jax.ShapeDtypeStruct((C * G * R * pad,), jnp.float32),
        scratch_types=[
            pltpu.VMEM((R * pad // 2,), jnp.int32),
            pltpu.VMEM((QS,), jnp.float32),
            pltpu.VMEM((QS,), jnp.float32),
            pltpu.VMEM((R * pad,), jnp.float32),
            pltpu.VMEM((R * pad,), jnp.float32),
            pltpu.SemaphoreType.DMA,
            pltpu.SemaphoreType.DMA,
            pltpu.SemaphoreType.DMA,
            pltpu.SemaphoreType.DMA,
        ],
        **_CPARAMS,
    )
    def pass1(x_hbm, rho_hbm, inter_hbm, rho_v, a0, a1, b0, b1,
              sa0, sa1, sb0, sb1):
        wid = lax.axis_index("s") * 2 + lax.axis_index("c")
        g = wid % G
        par = wid // G
        pltpu.sync_copy(
            rho_hbm.at[pl.ds(g * (R * pad // 2), R * pad // 2)], rho_v
        )
        npairs = C // 2

        def in_start(i, a_v, sem):
            c = jnp.minimum(i, npairs - 1) * 2 + par
            pltpu.async_copy(x_hbm.at[pl.ds(c * N + g * QS, QS)], a_v, sem)

        def in_wait(a_v, sem):
            pltpu.make_async_copy(
                x_hbm.at[pl.ds(g * QS, QS)], a_v, sem
            ).wait()

        def out_wait(b_v, sem):
            pltpu.make_async_copy(
                b_v, inter_hbm.at[pl.ds(0, R * pad)], sem
            ).wait()

        def shuffle_and_send(i, a_v, b_v, sem):
            c = i * 2 + par
            for rseg in range(R):
                base = rseg * pad

                def grp(j, carry2, base=base):
                    packed = rho_v[pl.ds((base // 2) + j * 16, 16)]
                    lo = lax.bitwise_and(packed, 0xFFFF)
                    hi = lax.shift_right_logical(packed, 16)
                    b_v[pl.ds(base + j * 32, 16)] = plsc.load_gather(
                        a_v, [lo]
                    )
                    b_v[pl.ds(base + j * 32 + 16, 16)] = plsc.load_gather(
                        a_v, [hi]
                    )
                    return carry2

                lax.fori_loop(0, pad // 32, grp, 0, unroll=6)
                pltpu.async_copy(
                    b_v.at[pl.ds(base, pad)],
                    inter_hbm.at[pl.ds(((c * R + rseg) * G + g) * pad, pad)],
                    sem,
                )

        in_start(0, a0, sa0)

        def step(i2, carry):
            for bsel, a_v, b_v, sa, sb in (
                (0, a0, b0, sa0, sb0),
                (1, a1, b1, sa1, sb1),
            ):
                i = i2 * 2 + bsel
                in_wait(a_v, sa)
                in_start(i + 1, (a1, a0)[bsel], (sa1, sa0)[bsel])

                @pl.when(i2 > 0)
                def _():
                    out_wait(b_v, sb)

                shuffle_and_send(i, a_v, b_v, sb)
            return carry

        lax.fori_loop(0, npairs // 2, step, 0)
        in_wait(a0, sa0)
        out_wait(b0, sb0)
        out_wait(b1, sb1)

    return pass1


def _make_pass2(pad):
    mesh = plsc.VectorSubcoreMesh(core_axis_name="c", subcore_axis_name="s")

    @functools.partial(
        pl.kernel,
        mesh=mesh,
        out_type=jax.ShapeDtypeStruct((C * N,), jnp.float32),
        scratch_types=[
            pltpu.VMEM((QD // 2,), jnp.int32),
            pltpu.VMEM((G * pad,), jnp.float32),
            pltpu.VMEM((G * pad,), jnp.float32),
            pltpu.VMEM((QD,), jnp.float32),
            pltpu.VMEM((QD,), jnp.float32),
            pltpu.SemaphoreType.DMA,
            pltpu.SemaphoreType.DMA,
            pltpu.SemaphoreType.DMA,
            pltpu.SemaphoreType.DMA,
        ],
        **_CPARAMS,
    )
    def pass2(inter_hbm, sigma_hbm, out_hbm, sig_v, rb0, rb1, o0, o1,
              si0, si1, so0, so1):
        wid = lax.axis_index("s") * 2 + lax.axis_index("c")
        r = wid % R
        par = wid // R
        pltpu.sync_copy(sigma_hbm.at[pl.ds(r * (QD // 2), QD // 2)], sig_v)
        npairs = C // 2

        def seg_start(i, rb_v, sem):
            c = jnp.minimum(i, npairs - 1) * 2 + par
            pltpu.async_copy(
                inter_hbm.at[pl.ds((c * R + r) * (G * pad), G * pad)],
                rb_v,
                sem,
            )

        def seg_wait(rb_v, sem):
            pltpu.make_async_copy(
                inter_hbm.at[pl.ds(0, G * pad)], rb_v, sem
            ).wait()

        def out_start(i, o_v, sem):
            c = i * 2 + par
            pltpu.async_copy(o_v, out_hbm.at[pl.ds(c * N + r * QD, QD)], sem)

        def out_wait(o_v, sem):
            pltpu.make_async_copy(
                o_v, out_hbm.at[pl.ds(r * QD, QD)], sem
            ).wait()

        def gather(rb_v, o_v):
            def grp(j, carry2):
                packed = sig_v[pl.ds(j * 16, 16)]
                lo = lax.bitwise_and(packed, 0xFFFF)
                hi = lax.shift_right_logical(packed, 16)
                o_v[pl.ds(j * 32, 16)] = plsc.load_gather(rb_v, [lo])
                o_v[pl.ds(j * 32 + 16, 16)] = plsc.load_gather(rb_v, [hi])
                return carry2

            lax.fori_loop(0, QD // 32, grp, 0, unroll=8)

        seg_start(0, rb0, si0)

        def step(i2, carry):
            for bsel, rb_v, o_v, si, so in (
                (0, rb0, o0, si0, so0),
                (1, rb1, o1, si1, so1),
            ):
                i = i2 * 2 + bsel
                seg_wait(rb_v, si)
                seg_start(i + 1, (rb1, rb0)[bsel], (si1, si0)[bsel])

                @pl.when(i2 > 0)
                def _():
                    out_wait(o_v, so)

                gather(rb_v, o_v)
                out_start(i, o_v, so)
            return carry

        lax.fori_loop(0, npairs // 2, step, 0)
        seg_wait(rb0, si0)
        out_wait(o0, so0)
        out_wait(o1, so1)

    return pass2


def kernel(x):
    pad, rho, sigma = _consts()
    x1d = x.reshape(C * N)
    rho_j = jnp.asarray(rho.reshape(-1))
    sigma_j = jnp.asarray(sigma.reshape(-1))
    inter = _make_pass1(pad)(x1d, rho_j)
    out = _make_pass2(pad)(inter, sigma_j)
    return out.reshape(C, H, W)


# speedup vs baseline: 2.7375x; 2.7375x over previous
import functools

import numpy as np

import jax
import jax.numpy as jnp
from jax import lax
from jax.experimental import pallas as pl
from jax.experimental.pallas import tpu as pltpu
from jax.experimental.pallas import tpu_sc as plsc

C = 384
H = 512
W = 512
N = H * W
NW = 32
G = 16
R = 16
QS = N // G
QD = N // R

_CONSTS = None


def _threefry2x32(k0, k1, x0, x1):
    u32 = np.uint32
    ks = [u32(k0), u32(k1), u32(k0) ^ u32(k1) ^ u32(0x1BD11BDA)]
    x = [x0.astype(u32).copy(), x1.astype(u32).copy()]

    def rounds(x, rots):
        for r in rots:
            x[0] = x[0] + x[1]
            x[1] = (x[1] << u32(r)) | (x[1] >> u32(32 - r))
            x[1] = x[0] ^ x[1]
        return x

    rot0, rot1 = [13, 15, 26, 6], [17, 29, 16, 24]
    x[0] = x[0] + ks[0]
    x[1] = x[1] + ks[1]
    x = rounds(x, rot0)
    x[0] = x[0] + ks[1]
    x[1] = x[1] + ks[2] + u32(1)
    x = rounds(x, rot1)
    x[0] = x[0] + ks[2]
    x[1] = x[1] + ks[0] + u32(2)
    x = rounds(x, rot0)
    x[0] = x[0] + ks[0]
    x[1] = x[1] + ks[1] + u32(3)
    x = rounds(x, rot1)
    x[0] = x[0] + ks[1]
    x[1] = x[1] + ks[2] + u32(4)
    x = rounds(x, rot0)
    x[0] = x[0] + ks[2]
    x[1] = x[1] + ks[0] + u32(5)
    return x[0], x[1]


def _np_permutation(seed, n):
    u32 = np.uint32
    key = (u32(np.int64(seed) >> 32), u32(np.int64(seed) & 0xFFFFFFFF))
    x = np.arange(n, dtype=np.int64)
    num_rounds = int(np.ceil(3 * np.log(max(1, n)) / np.log(2**32 - 1)))
    for _ in range(num_rounds):
        b1, b2 = _threefry2x32(
            key[0], key[1], np.zeros(2, u32), np.arange(2, dtype=u32)
        )
        key, subkey = (b1[0], b2[0]), (b1[1], b2[1])
        c1, c2 = np.zeros(n, u32), np.arange(n, dtype=u32)
        s1, s2 = _threefry2x32(subkey[0], subkey[1], c1, c2)
        x = x[np.argsort(s1 ^ s2, kind="stable")]
    return x


def _pack_pairs(idx):
    m, q = idx.shape
    pairs = idx.reshape(m, q // 32, 2, 16)
    return (pairs[:, :, 0, :] | (pairs[:, :, 1, :] << 16)).astype(
        np.int32
    ).reshape(m, q // 2)


def _consts():
    global _CONSTS
    if _CONSTS is None:
        perm = _np_permutation(0, N)
        inv = np.empty(N, np.int64)
        inv[perm] = np.arange(N)
        g_of = np.arange(N) // QS
        r_of = inv // QD
        key = g_of * R + r_of
        counts = np.bincount(key, minlength=G * R)
        starts = np.concatenate(([0], np.cumsum(counts)[:-1]))
        order = np.argsort(key, kind="stable")
        ranks = np.empty(N, np.int64)
        ranks[order] = np.arange(N) - np.repeat(starts, counts)
        pad = int(-(-counts.max() // 32) * 32)
        assert R * pad < 65536 and G * pad < 65536
        rho = (r_of * pad + ranks).reshape(G, QS)
        sigma = (g_of[perm] * pad + ranks[perm]).reshape(R, QD)
        _CONSTS = (pad, _pack_pairs(rho), _pack_pairs(sigma))
    return _CONSTS


_CPARAMS = dict(
    compiler_params=pltpu.CompilerParams(needs_layout_passes=False),
)


def _make_pass1(pad):
    mesh = plsc.VectorSubcoreMesh(core_axis_name="c", subcore_axis_name="s")

    @functools.partial(
        pl.kernel,
        mesh=mesh,
        out_type=jax.ShapeDtypeStruct((C * G * R * pad,), jnp.float32),
        scratch_types=[
            pltpu.VMEM((QS // 2,), jnp.int32),
            pltpu.VMEM((QS,), jnp.float32),
            pltpu.VMEM((QS,), jnp.float32),
            pltpu.VMEM((R * pad,), jnp.float32),
            pltpu.VMEM((R * pad,), jnp.float32),
            pltpu.SemaphoreType.DMA,
            pltpu.SemaphoreType.DMA,
            pltpu.SemaphoreType.DMA,
            pltpu.SemaphoreType.DMA,
        ],
        **_CPARAMS,
    )
    def pass1(x_hbm, rho_hbm, inter_hbm, rho_v, a0, a1, b0, b1,
              sa0, sa1, sb0, sb1):
        wid = lax.axis_index("s") * 2 + lax.axis_index("c")
        g = wid % G
        par = wid // G
        pltpu.sync_copy(rho_hbm.at[pl.ds(g * (QS // 2), QS // 2)], rho_v)
        npairs = C // 2

        def in_start(i, a_v, sem):
            c = jnp.minimum(i, npairs - 1) * 2 + par
            pltpu.async_copy(x_hbm.at[pl.ds(c * N + g * QS, QS)], a_v, sem)

        def in_wait(a_v, sem):
            pltpu.make_async_copy(
                x_hbm.at[pl.ds(g * QS, QS)], a_v, sem
            ).wait()

        def out_start(i, b_v, sem):
            c = i * 2 + par
            pltpu.async_copy(
                b_v,
                inter_hbm.at[pl.ds((c * G + g) * (R * pad), R * pad)],
                sem,
            )

        def out_wait(b_v, sem):
            pltpu.make_async_copy(
                b_v, inter_hbm.at[pl.ds(0, R * pad)], sem
            ).wait()

        def shuffle(a_v, b_v):
            @plsc.parallel_loop(0, QS // 32, unroll=8)
            def grp(j):
                packed = rho_v[pl.ds(j * 16, 16)]
                lo = lax.bitwise_and(packed, 0xFFFF)
                hi = lax.shift_right_logical(packed, 16)
                plsc.store_scatter(b_v, [lo], a_v[pl.ds(j * 32, 16)])
                plsc.store_scatter(b_v, [hi], a_v[pl.ds(j * 32 + 16, 16)])

        in_start(0, a0, sa0)

        def step(i2, carry):
            for bsel, a_v, b_v, sa, sb in (
                (0, a0, b0, sa0, sb0),
                (1, a1, b1, sa1, sb1),
            ):
                i = i2 * 2 + bsel
                in_wait(a_v, sa)
                in_start(i + 1, (a1, a0)[bsel], (sa1, sa0)[bsel])

                @pl.when(i2 > 0)
                def _():
                    out_wait(b_v, sb)

                shuffle(a_v, b_v)
                out_start(i, b_v, sb)
            return carry

        lax.fori_loop(0, npairs // 2, step, 0)
        in_wait(a0, sa0)
        out_wait(b0, sb0)
        out_wait(b1, sb1)

    return pass1


def _make_pass2(pad):
    mesh = plsc.VectorSubcoreMesh(core_axis_name="c", subcore_axis_name="s")

    @functools.partial(
        pl.kernel,
        mesh=mesh,
        out_type=jax.ShapeDtypeStruct((C * N,), jnp.float32),
        scratch_types=[
            pltpu.VMEM((QD // 2,), jnp.int32),
            pltpu.VMEM((G * pad,), jnp.float32),
            pltpu.VMEM((G * pad,), jnp.float32),
            pltpu.VMEM((QD,), jnp.float32),
            pltpu.VMEM((QD,), jnp.float32),
            pltpu.SemaphoreType.DMA,
            pltpu.SemaphoreType.DMA,
            pltpu.SemaphoreType.DMA,
            pltpu.SemaphoreType.DMA,
        ],
        **_CPARAMS,
    )
    def pass2(inter_hbm, sigma_hbm, out_hbm, sig_v, rb0, rb1, o0, o1,
              si0, si1, so0, so1):
        wid = lax.axis_index("s") * 2 + lax.axis_index("c")
        r = wid % R
        par = wid // R
        pltpu.sync_copy(sigma_hbm.at[pl.ds(r * (QD // 2), QD // 2)], sig_v)
        npairs = C // 2

        def seg_start(i, rb_v, sem):
            c = jnp.minimum(i, npairs - 1) * 2 + par
            for gp in range(G):
                pltpu.async_copy(
                    inter_hbm.at[pl.ds(((c * G + gp) * R + r) * pad, pad)],
                    rb_v.at[pl.ds(gp * pad, pad)],
                    sem,
                )

        def seg_wait(rb_v, sem):
            pltpu.make_async_copy(
                inter_hbm.at[pl.ds(0, G * pad)], rb_v, sem
            ).wait()

        def out_start(i, o_v, sem):
            c = i * 2 + par
            pltpu.async_copy(o_v, out_hbm.at[pl.ds(c * N + r * QD, QD)], sem)

        def out_wait(o_v, sem):
            pltpu.make_async_copy(
                o_v, out_hbm.at[pl.ds(r * QD, QD)], sem
            ).wait()

        def gather(rb_v, o_v):
            @plsc.parallel_loop(0, QD // 32, unroll=8)
            def grp(j):
                packed = sig_v[pl.ds(j * 16, 16)]
                lo = lax.bitwise_and(packed, 0xFFFF)
                hi = lax.shift_right_logical(packed, 16)
                o_v[pl.ds(j * 32, 16)] = plsc.load_gather(rb_v, [lo])
                o_v[pl.ds(j * 32 + 16, 16)] = plsc.load_gather(rb_v, [hi])

        seg_start(0, rb0, si0)

        def step(i2, carry):
            for bsel, rb_v, o_v, si, so in (
                (0, rb0, o0, si0, so0),
                (1, rb1, o1, si1, so1),
            ):
                i = i2 * 2 + bsel
                seg_wait(rb_v, si)
                seg_start(i + 1, (rb1, rb0)[bsel], (si1, si0)[bsel])

                @pl.when(i2 > 0)
                def _():
                    out_wait(o_v, so)

                gather(rb_v, o_v)
                out_start(i, o_v, so)
            return carry

        lax.fori_loop(0, npairs // 2, step, 0)
        seg_wait(rb0, si0)
        out_wait(o0, so0)
        out_wait(o1, so1)

    return pass2


def kernel(x):
    pad, rho, sigma = _consts()
    x1d = x.reshape(C * N)
    rho_j = jnp.asarray(rho.reshape(-1))
    sigma_j = jnp.asarray(sigma.reshape(-1))
    inter = _make_pass1(pad)(x1d, rho_j)
    out = _make_pass2(pad)(inter, sigma_j)
    return out.reshape(C, H, W)


# speedup vs baseline: 3.8709x; 1.4140x over previous
import functools

import numpy as np

import jax
import jax.numpy as jnp
from jax import lax
from jax.experimental import pallas as pl
from jax.experimental.pallas import tpu as pltpu
from jax.experimental.pallas import tpu_sc as plsc

C = 384
H = 512
W = 512
N = H * W
NW = 32
G = 16
R = 16
QS = N // G
QD = N // R

_CONSTS = None


def _threefry2x32(k0, k1, x0, x1):
    u32 = np.uint32
    ks = [u32(k0), u32(k1), u32(k0) ^ u32(k1) ^ u32(0x1BD11BDA)]
    x = [x0.astype(u32).copy(), x1.astype(u32).copy()]

    def rounds(x, rots):
        for r in rots:
            x[0] = x[0] + x[1]
            x[1] = (x[1] << u32(r)) | (x[1] >> u32(32 - r))
            x[1] = x[0] ^ x[1]
        return x

    rot0, rot1 = [13, 15, 26, 6], [17, 29, 16, 24]
    x[0] = x[0] + ks[0]
    x[1] = x[1] + ks[1]
    x = rounds(x, rot0)
    x[0] = x[0] + ks[1]
    x[1] = x[1] + ks[2] + u32(1)
    x = rounds(x, rot1)
    x[0] = x[0] + ks[2]
    x[1] = x[1] + ks[0] + u32(2)
    x = rounds(x, rot0)
    x[0] = x[0] + ks[0]
    x[1] = x[1] + ks[1] + u32(3)
    x = rounds(x, rot1)
    x[0] = x[0] + ks[1]
    x[1] = x[1] + ks[2] + u32(4)
    x = rounds(x, rot0)
    x[0] = x[0] + ks[2]
    x[1] = x[1] + ks[0] + u32(5)
    return x[0], x[1]


def _np_permutation(seed, n):
    u32 = np.uint32
    key = (u32(np.int64(seed) >> 32), u32(np.int64(seed) & 0xFFFFFFFF))
    x = np.arange(n, dtype=np.int64)
    num_rounds = int(np.ceil(3 * np.log(max(1, n)) / np.log(2**32 - 1)))
    for _ in range(num_rounds):
        b1, b2 = _threefry2x32(
            key[0], key[1], np.zeros(2, u32), np.arange(2, dtype=u32)
        )
        key, subkey = (b1[0], b2[0]), (b1[1], b2[1])
        c1, c2 = np.zeros(n, u32), np.arange(n, dtype=u32)
        s1, s2 = _threefry2x32(subkey[0], subkey[1], c1, c2)
        x = x[np.argsort(s1 ^ s2, kind="stable")]
    return x


def _pack_pairs(idx):
    m, q = idx.shape
    pairs = idx.reshape(m, q // 32, 2, 16)
    return (pairs[:, :, 0, :] | (pairs[:, :, 1, :] << 16)).astype(
        np.int32
    ).reshape(m, q // 2)


def _consts():
    global _CONSTS
    if _CONSTS is None:
        perm = _np_permutation(0, N)
        inv = np.empty(N, np.int64)
        inv[perm] = np.arange(N)
        g_of = np.arange(N) // QS
        r_of = inv // QD
        key = g_of * R + r_of
        counts = np.bincount(key, minlength=G * R)
        starts = np.concatenate(([0], np.cumsum(counts)[:-1]))
        order = np.argsort(key, kind="stable")
        ranks = np.empty(N, np.int64)
        ranks[order] = np.arange(N) - np.repeat(starts, counts)
        pad = int(-(-counts.max() // 32) * 32)
        assert R * pad < 65536 and G * pad < 65536
        rho = (r_of * pad + ranks).reshape(G, QS)
        sigma = (g_of[perm] * pad + ranks[perm]).reshape(R, QD)
        _CONSTS = (pad, _pack_pairs(rho), _pack_pairs(sigma))
    return _CONSTS


_CPARAMS = dict(
    compiler_params=pltpu.CompilerParams(needs_layout_passes=False),
)


def _make_pass1(pad):
    mesh = plsc.VectorSubcoreMesh(core_axis_name="c", subcore_axis_name="s")

    @functools.partial(
        pl.kernel,
        mesh=mesh,
        out_type=jax.ShapeDtypeStruct((C * G * R * pad,), jnp.float32),
        scratch_types=[
            pltpu.VMEM((QS // 2,), jnp.int32),
            pltpu.VMEM((QS,), jnp.float32),
            pltpu.VMEM((QS,), jnp.float32),
            pltpu.VMEM((R * pad,), jnp.float32),
            pltpu.VMEM((R * pad,), jnp.float32),
            pltpu.SemaphoreType.DMA,
            pltpu.SemaphoreType.DMA,
            pltpu.SemaphoreType.DMA,
            pltpu.SemaphoreType.DMA,
        ],
        **_CPARAMS,
    )
    def pass1(x_hbm, rho_hbm, inter_hbm, rho_v, a0, a1, b0, b1,
              sa0, sa1, sb0, sb1):
        wid = lax.axis_index("s") * 2 + lax.axis_index("c")
        g = wid % G
        par = wid // G
        pltpu.sync_copy(rho_hbm.at[pl.ds(g * (QS // 2), QS // 2)], rho_v)
        npairs = C // 2

        def in_start(i, a_v, sem):
            c = jnp.minimum(i, npairs - 1) * 2 + par
            pltpu.async_copy(x_hbm.at[pl.ds(c * N + g * QS, QS)], a_v, sem)

        def in_wait(a_v, sem):
            pltpu.make_async_copy(
                x_hbm.at[pl.ds(g * QS, QS)], a_v, sem
            ).wait()

        def out_start(i, b_v, sem):
            c = i * 2 + par
            pltpu.async_copy(
                b_v,
                inter_hbm.at[pl.ds((c * G + g) * (R * pad), R * pad)],
                sem,
            )

        def out_wait(b_v, sem):
            pltpu.make_async_copy(
                b_v, inter_hbm.at[pl.ds(0, R * pad)], sem
            ).wait()

        def shuffle(a_v, b_v):
            @plsc.parallel_loop(0, QS // 32, unroll=8)
            def grp(j):
                packed = rho_v[pl.ds(j * 16, 16)]
                lo = lax.bitwise_and(packed, 0xFFFF)
                hi = lax.shift_right_logical(packed, 16)
                plsc.store_scatter(b_v, [lo], a_v[pl.ds(j * 32, 16)])
                plsc.store_scatter(b_v, [hi], a_v[pl.ds(j * 32 + 16, 16)])

        in_start(0, a0, sa0)

        def step(i2, carry):
            for bsel, a_v, b_v, sa, sb in (
                (0, a0, b0, sa0, sb0),
                (1, a1, b1, sa1, sb1),
            ):
                i = i2 * 2 + bsel
                in_wait(a_v, sa)
                in_start(i + 1, (a1, a0)[bsel], (sa1, sa0)[bsel])

                @pl.when(i2 > 0)
                def _():
                    out_wait(b_v, sb)

                shuffle(a_v, b_v)
                out_start(i, b_v, sb)
            return carry

        lax.fori_loop(0, npairs // 2, step, 0)
        in_wait(a0, sa0)
        out_wait(b0, sb0)
        out_wait(b1, sb1)

    return pass1


def _make_pass2(pad):
    mesh = plsc.VectorSubcoreMesh(core_axis_name="c", subcore_axis_name="s")

    @functools.partial(
        pl.kernel,
        mesh=mesh,
        out_type=jax.ShapeDtypeStruct((C, H, W), jnp.float32),
        scratch_types=[
            pltpu.VMEM((QD // 2,), jnp.int32),
            pltpu.VMEM((G * pad,), jnp.float32),
            pltpu.VMEM((G * pad,), jnp.float32),
            pltpu.VMEM((QD // W, W), jnp.float32),
            pltpu.VMEM((QD // W, W), jnp.float32),
            pltpu.SemaphoreType.DMA,
            pltpu.SemaphoreType.DMA,
            pltpu.SemaphoreType.DMA,
            pltpu.SemaphoreType.DMA,
        ],
        **_CPARAMS,
    )
    def pass2(inter_hbm, sigma_hbm, out_hbm, sig_v, rb0, rb1, o0, o1,
              si0, si1, so0, so1):
        wid = lax.axis_index("s") * 2 + lax.axis_index("c")
        r = wid % R
        par = wid // R
        pltpu.sync_copy(sigma_hbm.at[pl.ds(r * (QD // 2), QD // 2)], sig_v)
        npairs = C // 2

        def seg_start(i, rb_v, sem):
            c = jnp.minimum(i, npairs - 1) * 2 + par
            for gp in range(G):
                pltpu.async_copy(
                    inter_hbm.at[pl.ds(((c * G + gp) * R + r) * pad, pad)],
                    rb_v.at[pl.ds(gp * pad, pad)],
                    sem,
                )

        def seg_wait(rb_v, sem):
            pltpu.make_async_copy(
                inter_hbm.at[pl.ds(0, G * pad)], rb_v, sem
            ).wait()

        rows = QD // W

        def out_start(i, o_v, sem):
            c = i * 2 + par
            pltpu.async_copy(
                o_v, out_hbm.at[c, pl.ds(r * rows, rows), :], sem
            )

        def out_wait(o_v, sem):
            pltpu.make_async_copy(
                o_v, out_hbm.at[0, pl.ds(0, rows), :], sem
            ).wait()

        gpr = W // 32

        def gather(rb_v, o_v):
            @plsc.parallel_loop(0, QD // 32, unroll=8)
            def grp(j):
                packed = sig_v[pl.ds(j * 16, 16)]
                lo = lax.bitwise_and(packed, 0xFFFF)
                hi = lax.shift_right_logical(packed, 16)
                row = j // gpr
                colbase = (j % gpr) * 32
                o_v[row, pl.ds(colbase, 16)] = plsc.load_gather(rb_v, [lo])
                o_v[row, pl.ds(colbase + 16, 16)] = plsc.load_gather(
                    rb_v, [hi]
                )

        seg_start(0, rb0, si0)

        def step(i2, carry):
            for bsel, rb_v, o_v, si, so in (
                (0, rb0, o0, si0, so0),
                (1, rb1, o1, si1, so1),
            ):
                i = i2 * 2 + bsel
                seg_wait(rb_v, si)
                seg_start(i + 1, (rb1, rb0)[bsel], (si1, si0)[bsel])

                @pl.when(i2 > 0)
                def _():
                    out_wait(o_v, so)

                gather(rb_v, o_v)
                out_start(i, o_v, so)
            return carry

        lax.fori_loop(0, npairs // 2, step, 0)
        seg_wait(rb0, si0)
        out_wait(o0, so0)
        out_wait(o1, so1)

    return pass2


def kernel(x):
    pad, rho, sigma = _consts()
    x1d = x.reshape(C * N)
    rho_j = jnp.asarray(rho.reshape(-1))
    sigma_j = jnp.asarray(sigma.reshape(-1))
    inter = _make_pass1(pad)(x1d, rho_j)
    return _make_pass2(pad)(inter, sigma_j)


# speedup vs baseline: 5.4299x; 1.4028x over previous
import functools

import numpy as np

import jax
import jax.numpy as jnp
from jax import lax
from jax.experimental import pallas as pl
from jax.experimental.pallas import tpu as pltpu
from jax.experimental.pallas import tpu_sc as plsc

C = 384
H = 512
W = 512
N = H * W
NW = 32
G = 16
R = 16
QS = N // G
QD = N // R

_CONSTS = None


def _threefry2x32(k0, k1, x0, x1):
    u32 = np.uint32
    ks = [u32(k0), u32(k1), u32(k0) ^ u32(k1) ^ u32(0x1BD11BDA)]
    x = [x0.astype(u32).copy(), x1.astype(u32).copy()]

    def rounds(x, rots):
        for r in rots:
            x[0] = x[0] + x[1]
            x[1] = (x[1] << u32(r)) | (x[1] >> u32(32 - r))
            x[1] = x[0] ^ x[1]
        return x

    rot0, rot1 = [13, 15, 26, 6], [17, 29, 16, 24]
    x[0] = x[0] + ks[0]
    x[1] = x[1] + ks[1]
    x = rounds(x, rot0)
    x[0] = x[0] + ks[1]
    x[1] = x[1] + ks[2] + u32(1)
    x = rounds(x, rot1)
    x[0] = x[0] + ks[2]
    x[1] = x[1] + ks[0] + u32(2)
    x = rounds(x, rot0)
    x[0] = x[0] + ks[0]
    x[1] = x[1] + ks[1] + u32(3)
    x = rounds(x, rot1)
    x[0] = x[0] + ks[1]
    x[1] = x[1] + ks[2] + u32(4)
    x = rounds(x, rot0)
    x[0] = x[0] + ks[2]
    x[1] = x[1] + ks[0] + u32(5)
    return x[0], x[1]


def _np_permutation(seed, n):
    u32 = np.uint32
    key = (u32(np.int64(seed) >> 32), u32(np.int64(seed) & 0xFFFFFFFF))
    x = np.arange(n, dtype=np.int64)
    num_rounds = int(np.ceil(3 * np.log(max(1, n)) / np.log(2**32 - 1)))
    for _ in range(num_rounds):
        b1, b2 = _threefry2x32(
            key[0], key[1], np.zeros(2, u32), np.arange(2, dtype=u32)
        )
        key, subkey = (b1[0], b2[0]), (b1[1], b2[1])
        c1, c2 = np.zeros(n, u32), np.arange(n, dtype=u32)
        s1, s2 = _threefry2x32(subkey[0], subkey[1], c1, c2)
        x = x[np.argsort(s1 ^ s2, kind="stable")]
    return x


def _pack_pairs(idx):
    m, q = idx.shape
    pairs = idx.reshape(m, q // 32, 2, 16)
    return (pairs[:, :, 0, :] | (pairs[:, :, 1, :] << 16)).astype(
        np.int32
    ).reshape(m, q // 2)


def _consts():
    global _CONSTS
    if _CONSTS is None:
        perm = _np_permutation(0, N)
        inv = np.empty(N, np.int64)
        inv[perm] = np.arange(N)
        g_of = np.arange(N) // QS
        r_of = inv // QD
        key = g_of * R + r_of
        counts = np.bincount(key, minlength=G * R)
        starts = np.concatenate(([0], np.cumsum(counts)[:-1]))
        order = np.argsort(key, kind="stable")
        ranks = np.empty(N, np.int64)
        ranks[order] = np.arange(N) - np.repeat(starts, counts)
        pad = int(-(-counts.max() // 32) * 32)
        assert R * pad < 65536 and G * pad < 65536
        rho = (r_of * pad + ranks).reshape(G, QS)
        sigma = (g_of[perm] * pad + ranks[perm]).reshape(R, QD)
        _CONSTS = (pad, _pack_pairs(rho), _pack_pairs(sigma))
    return _CONSTS


_CPARAMS = dict(
    compiler_params=pltpu.CompilerParams(needs_layout_passes=False),
)


def _make_pass1(pad):
    mesh = plsc.VectorSubcoreMesh(core_axis_name="c", subcore_axis_name="s")

    @functools.partial(
        pl.kernel,
        mesh=mesh,
        out_type=jax.ShapeDtypeStruct((C * G * R * pad,), jnp.float32),
        scratch_types=[
            pltpu.VMEM((QS // 2,), jnp.int32),
            pltpu.VMEM((QS // W, W), jnp.float32),
            pltpu.VMEM((QS // W, W), jnp.float32),
            pltpu.VMEM((R * pad,), jnp.float32),
            pltpu.VMEM((R * pad,), jnp.float32),
            pltpu.SemaphoreType.DMA,
            pltpu.SemaphoreType.DMA,
            pltpu.SemaphoreType.DMA,
            pltpu.SemaphoreType.DMA,
        ],
        **_CPARAMS,
    )
    def pass1(x_hbm, rho_hbm, inter_hbm, rho_v, a0, a1, b0, b1,
              sa0, sa1, sb0, sb1):
        wid = lax.axis_index("s") * 2 + lax.axis_index("c")
        g = wid % G
        par = wid // G
        pltpu.sync_copy(rho_hbm.at[pl.ds(g * (QS // 2), QS // 2)], rho_v)
        npairs = C // 2

        srows = QS // W

        def in_start(i, a_v, sem):
            c = jnp.minimum(i, npairs - 1) * 2 + par
            pltpu.async_copy(
                x_hbm.at[c, pl.ds(g * srows, srows), :], a_v, sem
            )

        def in_wait(a_v, sem):
            pltpu.make_async_copy(
                x_hbm.at[0, pl.ds(0, srows), :], a_v, sem
            ).wait()

        def out_start(i, b_v, sem):
            c = i * 2 + par
            pltpu.async_copy(
                b_v,
                inter_hbm.at[pl.ds((c * G + g) * (R * pad), R * pad)],
                sem,
            )

        def out_wait(b_v, sem):
            pltpu.make_async_copy(
                b_v, inter_hbm.at[pl.ds(0, R * pad)], sem
            ).wait()

        gpr = W // 32

        def shuffle(a_v, b_v):
            @plsc.parallel_loop(0, QS // 32, unroll=8)
            def grp(j):
                packed = rho_v[pl.ds(j * 16, 16)]
                lo = lax.bitwise_and(packed, 0xFFFF)
                hi = lax.shift_right_logical(packed, 16)
                row = j // gpr
                colbase = (j % gpr) * 32
                plsc.store_scatter(
                    b_v, [lo], a_v[row, pl.ds(colbase, 16)]
                )
                plsc.store_scatter(
                    b_v, [hi], a_v[row, pl.ds(colbase + 16, 16)]
                )

        in_start(0, a0, sa0)

        def step(i2, carry):
            for bsel, a_v, b_v, sa, sb in (
                (0, a0, b0, sa0, sb0),
                (1, a1, b1, sa1, sb1),
            ):
                i = i2 * 2 + bsel
                in_wait(a_v, sa)
                in_start(i + 1, (a1, a0)[bsel], (sa1, sa0)[bsel])

                @pl.when(i2 > 0)
                def _():
                    out_wait(b_v, sb)

                shuffle(a_v, b_v)
                out_start(i, b_v, sb)
            return carry

        lax.fori_loop(0, npairs // 2, step, 0)
        in_wait(a0, sa0)
        out_wait(b0, sb0)
        out_wait(b1, sb1)

    return pass1


def _make_pass2(pad):
    mesh = plsc.VectorSubcoreMesh(core_axis_name="c", subcore_axis_name="s")

    @functools.partial(
        pl.kernel,
        mesh=mesh,
        out_type=jax.ShapeDtypeStruct((C, H, W), jnp.float32),
        scratch_types=[
            pltpu.VMEM((QD // 2,), jnp.int32),
            pltpu.VMEM((G * pad,), jnp.float32),
            pltpu.VMEM((G * pad,), jnp.float32),
            pltpu.VMEM((QD // W, W), jnp.float32),
            pltpu.VMEM((QD // W, W), jnp.float32),
            pltpu.SemaphoreType.DMA,
            pltpu.SemaphoreType.DMA,
            pltpu.SemaphoreType.DMA,
            pltpu.SemaphoreType.DMA,
        ],
        **_CPARAMS,
    )
    def pass2(inter_hbm, sigma_hbm, out_hbm, sig_v, rb0, rb1, o0, o1,
              si0, si1, so0, so1):
        wid = lax.axis_index("s") * 2 + lax.axis_index("c")
        r = wid % R
        par = wid // R
        pltpu.sync_copy(sigma_hbm.at[pl.ds(r * (QD // 2), QD // 2)], sig_v)
        npairs = C // 2

        def seg_start(i, rb_v, sem):
            c = jnp.minimum(i, npairs - 1) * 2 + par
            for gp in range(G):
                pltpu.async_copy(
                    inter_hbm.at[pl.ds(((c * G + gp) * R + r) * pad, pad)],
                    rb_v.at[pl.ds(gp * pad, pad)],
                    sem,
                )

        def seg_wait(rb_v, sem):
            pltpu.make_async_copy(
                inter_hbm.at[pl.ds(0, G * pad)], rb_v, sem
            ).wait()

        rows = QD // W

        def out_start(i, o_v, sem):
            c = i * 2 + par
            pltpu.async_copy(
                o_v, out_hbm.at[c, pl.ds(r * rows, rows), :], sem
            )

        def out_wait(o_v, sem):
            pltpu.make_async_copy(
                o_v, out_hbm.at[0, pl.ds(0, rows), :], sem
            ).wait()

        gpr = W // 32

        def gather(rb_v, o_v):
            @plsc.parallel_loop(0, QD // 32, unroll=8)
            def grp(j):
                packed = sig_v[pl.ds(j * 16, 16)]
                lo = lax.bitwise_and(packed, 0xFFFF)
                hi = lax.shift_right_logical(packed, 16)
                row = j // gpr
                colbase = (j % gpr) * 32
                o_v[row, pl.ds(colbase, 16)] = plsc.load_gather(rb_v, [lo])
                o_v[row, pl.ds(colbase + 16, 16)] = plsc.load_gather(
                    rb_v, [hi]
                )

        seg_start(0, rb0, si0)

        def step(i2, carry):
            for bsel, rb_v, o_v, si, so in (
                (0, rb0, o0, si0, so0),
                (1, rb1, o1, si1, so1),
            ):
                i = i2 * 2 + bsel
                seg_wait(rb_v, si)
                seg_start(i + 1, (rb1, rb0)[bsel], (si1, si0)[bsel])

                @pl.when(i2 > 0)
                def _():
                    out_wait(o_v, so)

                gather(rb_v, o_v)
                out_start(i, o_v, so)
            return carry

        lax.fori_loop(0, npairs // 2, step, 0)
        seg_wait(rb0, si0)
        out_wait(o0, so0)
        out_wait(o1, so1)

    return pass2


def kernel(x):
    pad, rho, sigma = _consts()
    rho_j = jnp.asarray(rho.reshape(-1))
    sigma_j = jnp.asarray(sigma.reshape(-1))
    inter = _make_pass1(pad)(x, rho_j)
    return _make_pass2(pad)(inter, sigma_j)
